# baseline (speedup 1.0000x reference)
import functools

import jax
import jax.numpy as jnp
from jax import lax
from jax.experimental import pallas as pl
from jax.experimental.pallas import tpu as pltpu
from jax.experimental.pallas import tpu_sc as plsc

_SEQ = 1048576
_V = 25
_D = 25
_L = 16
_NC = 2
_NS = 16
_NW = _NC * _NS
_TOK_PER_W = _SEQ // _NW
_CHUNK = 512
_N_CHUNK = _TOK_PER_W // _CHUNK
_GROUPS = _CHUNK // _L

_mesh = plsc.VectorSubcoreMesh(core_axis_name="c", subcore_axis_name="s")


@functools.partial(
    pl.kernel,
    mesh=_mesh,
    out_type=jax.ShapeDtypeStruct((_D, _SEQ), jnp.float32),
    compiler_params=pltpu.CompilerParams(
        needs_layout_passes=False, use_tc_tiling_on_sc=True
    ),
    scratch_types=[
        pltpu.VMEM((_V * _D,), jnp.float32),
        pltpu.VMEM((_TOK_PER_W,), jnp.int32),
        pltpu.VMEM((8, _CHUNK), jnp.float32),
        pltpu.VMEM((8, _CHUNK), jnp.float32),
        pltpu.VMEM((8, _CHUNK), jnp.float32),
        pltpu.VMEM((1, _CHUNK), jnp.float32),
        pltpu.VMEM((8, _CHUNK), jnp.float32),
        pltpu.VMEM((8, _CHUNK), jnp.float32),
        pltpu.VMEM((8, _CHUNK), jnp.float32),
        pltpu.VMEM((1, _CHUNK), jnp.float32),
        pltpu.SemaphoreType.DMA,
    ],
)
def _embed(seq_hbm, table_hbm, out_hbm, table_v, idx_v,
           a0, a1, a2, a3, b0, b1, b2, b3, sem_o):
    wid = lax.axis_index("s") * _NC + lax.axis_index("c")
    tok0 = wid * _TOK_PER_W
    pltpu.sync_copy(table_hbm, table_v)
    pltpu.sync_copy(seq_hbm.at[pl.ds(tok0, _TOK_PER_W)], idx_v)
    bufsets = ((a0, a1, a2, a3), (b0, b1, b2, b3))

    def compute_chunk(c, bufs):
        @plsc.parallel_loop(0, _GROUPS, 1, unroll=1)
        def _(m):
            idx16 = idx_v[pl.ds(c * _CHUNK + m * _L, _L)]
            srow = idx16 * _D
            for j in range(_D):
                g, jr = divmod(j, 8)
                vals = plsc.load_gather(table_v, [srow + j])
                bufs[g][jr if g < 3 else 0, pl.ds(m * _L, _L)] = vals

    def put_chunk(c, bufs):
        t0 = tok0 + c * _CHUNK
        for g in range(3):
            dst = out_hbm.at[pl.ds(g * 8, 8), pl.ds(t0, _CHUNK)]
            pltpu.async_copy(bufs[g], dst, sem_o)
        dst = out_hbm.at[pl.ds(24, 1), pl.ds(t0, _CHUNK)]
        pltpu.async_copy(bufs[3], dst, sem_o)

    def wait_out(bufs):
        for g in range(3):
            dst = out_hbm.at[pl.ds(g * 8, 8), pl.ds(tok0, _CHUNK)]
            pltpu.make_async_copy(bufs[g], dst, sem_o).wait()
        dst = out_hbm.at[pl.ds(24, 1), pl.ds(tok0, _CHUNK)]
        pltpu.make_async_copy(bufs[3], dst, sem_o).wait()

    compute_chunk(0, bufsets[0])
    put_chunk(0, bufsets[0])
    compute_chunk(1, bufsets[1])
    put_chunk(1, bufsets[1])

    def pair_body(i, carry):
        c0 = 2 * i
        wait_out(bufsets[0])
        compute_chunk(c0, bufsets[0])
        put_chunk(c0, bufsets[0])
        wait_out(bufsets[1])
        compute_chunk(c0 + 1, bufsets[1])
        put_chunk(c0 + 1, bufsets[1])
        return carry

    lax.fori_loop(1, _N_CHUNK // 2, pair_body, 0)
    wait_out(bufsets[0])
    wait_out(bufsets[1])


def kernel(sequence, table):
    res = _embed(sequence, table.reshape(-1))
    return res.T


# speedup vs baseline: 1.0307x; 1.0307x over previous
import functools

import jax
import jax.numpy as jnp
from jax import lax
from jax.experimental import pallas as pl
from jax.experimental.pallas import tpu as pltpu
from jax.experimental.pallas import tpu_sc as plsc

_SEQ = 1048576
_V = 25
_D = 25
_L = 16
_NC = 2
_NS = 16
_NW = _NC * _NS
_TOK_PER_W = _SEQ // _NW
_CHUNK = 1024
_N_CHUNK = _TOK_PER_W // _CHUNK
_GROUPS = _CHUNK // _L

_mesh = plsc.VectorSubcoreMesh(core_axis_name="c", subcore_axis_name="s")


@functools.partial(
    pl.kernel,
    mesh=_mesh,
    out_type=jax.ShapeDtypeStruct((_D, _SEQ), jnp.float32),
    compiler_params=pltpu.CompilerParams(
        needs_layout_passes=False, use_tc_tiling_on_sc=True
    ),
    scratch_types=[
        pltpu.VMEM((_V * _D,), jnp.float32),
        pltpu.VMEM((_TOK_PER_W,), jnp.int32),
        pltpu.VMEM((8, _CHUNK), jnp.float32),
        pltpu.VMEM((8, _CHUNK), jnp.float32),
        pltpu.VMEM((8, _CHUNK), jnp.float32),
        pltpu.VMEM((1, _CHUNK), jnp.float32),
        pltpu.VMEM((8, _CHUNK), jnp.float32),
        pltpu.VMEM((8, _CHUNK), jnp.float32),
        pltpu.VMEM((8, _CHUNK), jnp.float32),
        pltpu.VMEM((1, _CHUNK), jnp.float32),
        pltpu.SemaphoreType.DMA,
    ],
)
def _embed(seq_hbm, table_hbm, out_hbm, table_v, idx_v,
           a0, a1, a2, a3, b0, b1, b2, b3, sem_o):
    wid = lax.axis_index("s") * _NC + lax.axis_index("c")
    tok0 = wid * _TOK_PER_W
    pltpu.sync_copy(table_hbm, table_v)
    pltpu.sync_copy(seq_hbm.at[pl.ds(tok0, _TOK_PER_W)], idx_v)
    bufsets = ((a0, a1, a2, a3), (b0, b1, b2, b3))

    def compute_chunk(c, bufs):
        @plsc.parallel_loop(0, _GROUPS, 1, unroll=1)
        def _(m):
            idx16 = idx_v[pl.ds(c * _CHUNK + m * _L, _L)]
            srow = idx16 * _D
            for j in range(_D):
                g, jr = divmod(j, 8)
                vals = plsc.load_gather(table_v, [srow + j])
                bufs[g][jr if g < 3 else 0, pl.ds(m * _L, _L)] = vals

    def put_chunk(c, bufs):
        t0 = tok0 + c * _CHUNK
        for g in range(3):
            dst = out_hbm.at[pl.ds(g * 8, 8), pl.ds(t0, _CHUNK)]
            pltpu.async_copy(bufs[g], dst, sem_o)
        dst = out_hbm.at[pl.ds(24, 1), pl.ds(t0, _CHUNK)]
        pltpu.async_copy(bufs[3], dst, sem_o)

    def wait_out(bufs):
        for g in range(3):
            dst = out_hbm.at[pl.ds(g * 8, 8), pl.ds(tok0, _CHUNK)]
            pltpu.make_async_copy(bufs[g], dst, sem_o).wait()
        dst = out_hbm.at[pl.ds(24, 1), pl.ds(tok0, _CHUNK)]
        pltpu.make_async_copy(bufs[3], dst, sem_o).wait()

    compute_chunk(0, bufsets[0])
    put_chunk(0, bufsets[0])
    compute_chunk(1, bufsets[1])
    put_chunk(1, bufsets[1])

    def pair_body(i, carry):
        c0 = 2 * i
        wait_out(bufsets[0])
        compute_chunk(c0, bufsets[0])
        put_chunk(c0, bufsets[0])
        wait_out(bufsets[1])
        compute_chunk(c0 + 1, bufsets[1])
        put_chunk(c0 + 1, bufsets[1])
        return carry

    lax.fori_loop(1, _N_CHUNK // 2, pair_body, 0)
    wait_out(bufsets[0])
    wait_out(bufsets[1])


def kernel(sequence, table):
    res = _embed(sequence, table.reshape(-1))
    return res.T


# speedup vs baseline: 1.0315x; 1.0008x over previous
import functools

import jax
import jax.numpy as jnp
from jax import lax
from jax.experimental import pallas as pl
from jax.experimental.pallas import tpu as pltpu
from jax.experimental.pallas import tpu_sc as plsc

_SEQ = 1048576
_V = 25
_D = 25
_L = 16
_NC = 2
_NS = 16
_NW = _NC * _NS
_TOK_PER_W = _SEQ // _NW
_CHUNK = 1024
_N_CHUNK = _TOK_PER_W // _CHUNK
_GROUPS = _CHUNK // _L

_mesh = plsc.VectorSubcoreMesh(core_axis_name="c", subcore_axis_name="s")


@functools.partial(
    pl.kernel,
    mesh=_mesh,
    out_type=jax.ShapeDtypeStruct((_D, _SEQ), jnp.float32),
    compiler_params=pltpu.CompilerParams(
        needs_layout_passes=False,
        use_tc_tiling_on_sc=True,
        disable_bounds_checks=True,
        disable_semaphore_checks=True,
    ),
    scratch_types=[
        pltpu.VMEM((_V * _D,), jnp.float32),
        pltpu.VMEM((_TOK_PER_W,), jnp.int32),
        pltpu.VMEM((8, _CHUNK), jnp.float32),
        pltpu.VMEM((8, _CHUNK), jnp.float32),
        pltpu.VMEM((8, _CHUNK), jnp.float32),
        pltpu.VMEM((1, _CHUNK), jnp.float32),
        pltpu.VMEM((8, _CHUNK), jnp.float32),
        pltpu.VMEM((8, _CHUNK), jnp.float32),
        pltpu.VMEM((8, _CHUNK), jnp.float32),
        pltpu.VMEM((1, _CHUNK), jnp.float32),
        pltpu.SemaphoreType.DMA,
    ],
)
def _embed(seq_hbm, table_hbm, out_hbm, table_v, idx_v,
           a0, a1, a2, a3, b0, b1, b2, b3, sem_o):
    wid = lax.axis_index("s") * _NC + lax.axis_index("c")
    tok0 = wid * _TOK_PER_W
    pltpu.sync_copy(table_hbm, table_v)
    pltpu.sync_copy(seq_hbm.at[pl.ds(tok0, _TOK_PER_W)], idx_v)
    bufsets = ((a0, a1, a2, a3), (b0, b1, b2, b3))

    def compute_chunk(c, bufs):
        @plsc.parallel_loop(0, _GROUPS, 1, unroll=1)
        def _(m):
            idx16 = idx_v[pl.ds(c * _CHUNK + m * _L, _L)]
            srow = idx16 * _D
            for j in range(_D):
                g, jr = divmod(j, 8)
                vals = plsc.load_gather(table_v, [srow + j])
                bufs[g][jr if g < 3 else 0, pl.ds(m * _L, _L)] = vals

    def put_chunk(c, bufs):
        t0 = tok0 + c * _CHUNK
        for g in range(3):
            dst = out_hbm.at[pl.ds(g * 8, 8), pl.ds(t0, _CHUNK)]
            pltpu.async_copy(bufs[g], dst, sem_o)
        dst = out_hbm.at[pl.ds(24, 1), pl.ds(t0, _CHUNK)]
        pltpu.async_copy(bufs[3], dst, sem_o)

    def wait_out(bufs):
        for g in range(3):
            dst = out_hbm.at[pl.ds(g * 8, 8), pl.ds(tok0, _CHUNK)]
            pltpu.make_async_copy(bufs[g], dst, sem_o).wait()
        dst = out_hbm.at[pl.ds(24, 1), pl.ds(tok0, _CHUNK)]
        pltpu.make_async_copy(bufs[3], dst, sem_o).wait()

    compute_chunk(0, bufsets[0])
    put_chunk(0, bufsets[0])
    compute_chunk(1, bufsets[1])
    put_chunk(1, bufsets[1])

    def pair_body(i, carry):
        c0 = 2 * i
        wait_out(bufsets[0])
        compute_chunk(c0, bufsets[0])
        put_chunk(c0, bufsets[0])
        wait_out(bufsets[1])
        compute_chunk(c0 + 1, bufsets[1])
        put_chunk(c0 + 1, bufsets[1])
        return carry

    lax.fori_loop(1, _N_CHUNK // 2, pair_body, 0)
    wait_out(bufsets[0])
    wait_out(bufsets[1])


def kernel(sequence, table):
    res = _embed(sequence, table.reshape(-1))
    return res.T
